# baseline (speedup 1.0000x reference)
import functools

import jax
import jax.numpy as jnp
from jax import lax
from jax.experimental import pallas as pl
from jax.experimental.pallas import tpu as pltpu
from jax.experimental.pallas import tpu_sc as plsc

_EMBED = 20
_NROWS = 22
_PACK = 280
_OFF_W = 40
_OFF_W3 = 240
_OFF_B = 260


def _sc_body(table_hbm, pack_hbm, o1_hbm, o2_hbm, pack_v, rows_v, o_v, sem):
    c = lax.axis_index("c")
    s = lax.axis_index("s")

    @pl.when(jnp.logical_and(c == 0, s == 0))
    def _():
        pltpu.sync_copy(pack_hbm, pack_v)
        lanes = lax.iota(jnp.int32, 16)
        i0a = plsc.load_gather(pack_v, [lanes]).astype(jnp.int32)
        i0b = plsc.load_gather(pack_v, [lanes + 4]).astype(jnp.int32)
        i1 = plsc.load_gather(pack_v, [lanes + 20]).astype(jnp.int32)
        copies = []
        for k in range(10):
            copies.append(pltpu.async_copy(
                table_hbm.at[pl.ds(i0a[k], 1), :],
                rows_v.at[pl.ds(k, 1), :], sem))
        for k in range(10):
            src = i0a[10 + k] if k < 6 else i0b[6 + k]
            copies.append(pltpu.async_copy(
                table_hbm.at[pl.ds(src, 1), :],
                rows_v.at[pl.ds(10 + k, 1), :], sem))
        copies.append(pltpu.async_copy(
            table_hbm.at[pl.ds(i1[0], 1), :], rows_v.at[pl.ds(20, 1), :], sem))
        copies.append(pltpu.async_copy(
            table_hbm.at[pl.ds(i1[1], 1), :], rows_v.at[pl.ds(21, 1), :], sem))

        jidx = jnp.minimum(lanes, 9)
        slot1 = jidx
        slot2 = 10 + jidx
        wbase = _OFF_W + jidx * _EMBED

        w3a = plsc.load_gather(pack_v, [lanes + _OFF_W3])
        w3b = plsc.load_gather(pack_v, [lanes + (_OFF_W3 + 4)])
        bv = plsc.load_gather(pack_v, [_OFF_B + jidx])
        ba = plsc.load_gather(pack_v, [lanes + _OFF_B])

        for cp in copies:
            cp.wait()

        s20 = jnp.full((16,), 20, jnp.int32)
        s21 = jnp.full((16,), 21, jnp.int32)
        e1a = plsc.load_gather(rows_v, [s20, lanes])
        e1b = plsc.load_gather(rows_v, [s20, lanes + 4])
        e2a = plsc.load_gather(rows_v, [s21, lanes])
        e2b = plsc.load_gather(rows_v, [s21, lanes + 4])

        acc1 = jnp.zeros((16,), jnp.float32)
        acc2 = jnp.zeros((16,), jnp.float32)
        acc3 = jnp.zeros((16,), jnp.float32)
        acc4 = jnp.zeros((16,), jnp.float32)
        for d in range(_EMBED):
            dvec = jnp.full((16,), d, jnp.int32)
            wv = plsc.load_gather(pack_v, [wbase + d])
            v3 = plsc.load_gather(rows_v, [slot1, dvec])
            v4 = plsc.load_gather(rows_v, [slot2, dvec])
            w3 = w3a[d] if d < 16 else w3b[d - 4]
            e1 = e1a[d] if d < 16 else e1b[d - 4]
            e2 = e2a[d] if d < 16 else e2b[d - 4]
            acc1 = acc1 + e1 * wv
            acc2 = acc2 + e2 * wv
            acc3 = acc3 + v3 * w3
            acc4 = acc4 + v4 * w3

        bias = bv + ba[10]
        o_v[0:16] = acc1 + acc3 + bias
        o_v[16:32] = acc2 + acc4 + bias
        pltpu.sync_copy(o_v.at[0:10], o1_hbm.at[0])
        pltpu.sync_copy(o_v.at[16:26], o2_hbm.at[0])


@functools.lru_cache(maxsize=1)
def _sc_call():
    return functools.partial(
        pl.kernel,
        mesh=plsc.VectorSubcoreMesh(core_axis_name="c", subcore_axis_name="s",
                                    num_cores=1, num_subcores=1),
        compiler_params=pltpu.CompilerParams(
            needs_layout_passes=False, use_tc_tiling_on_sc=True),
        out_type=[jax.ShapeDtypeStruct((1, 10), jnp.float32),
                  jax.ShapeDtypeStruct((1, 10), jnp.float32)],
        scratch_types=[
            pltpu.VMEM((_PACK,), jnp.float32),
            pltpu.VMEM((_NROWS, _EMBED), jnp.float32),
            pltpu.VMEM((32,), jnp.float32),
            pltpu.SemaphoreType.DMA,
        ],
    )(_sc_body)


def kernel(DPTD_name_1, DPTD_name_2, DPTD_word_1, DPTD_word_2,
           table, W, b, W3, b3):
    pack = jnp.concatenate([
        DPTD_word_1.astype(jnp.float32),
        DPTD_word_2.astype(jnp.float32),
        jnp.stack([jnp.asarray(DPTD_name_1, jnp.float32),
                   jnp.asarray(DPTD_name_2, jnp.float32)]),
        jnp.zeros((18,), jnp.float32),
        W.reshape(-1),
        W3.reshape(-1),
        b,
        b3,
        jnp.zeros((9,), jnp.float32),
    ])
    r1, r2 = _sc_call()(table, pack)
    return (r1, r2)
